# Initial kernel scaffold; baseline (speedup 1.0000x reference)
import jax
import jax.numpy as jnp
from jax.experimental import pallas as pl


def kernel(x, edge_index, batch, W1e, as1e, ad1e, b1e, W2e, as2e, ad2e, b2e, W3e, as3e, ad3e, b3e, Wfe, bfe, Wfd, bfd, W1d, as1d, ad1d, b1d, W2d, as2d, ad2d, b2d, W3d, as3d, ad3d, b3d):
    raise NotImplementedError("write your pallas kernel here")


# speedup vs baseline: 1.5449x; 1.5449x over previous
import functools

import jax
import jax.numpy as jnp
from jax.experimental import pallas as pl


def _fcd_body(z_ref, w_ref, b_ref, o_ref):
    o_ref[...] = (
        jnp.dot(z_ref[...], w_ref[...], preferred_element_type=jnp.float32)
        + b_ref[...]
    )


def _fcd_matvec(z, Wfd, bfd):
    K, M = Wfd.shape
    BC = 12800
    z8 = jnp.concatenate([z, jnp.zeros((7, K), z.dtype)], axis=0)
    out = pl.pallas_call(
        _fcd_body,
        grid=(M // BC,),
        in_specs=[
            pl.BlockSpec((8, K), lambda i: (0, 0)),
            pl.BlockSpec((K, BC), lambda i: (0, i)),
            pl.BlockSpec((1, BC), lambda i: (0, i)),
        ],
        out_specs=pl.BlockSpec((8, BC), lambda i: (0, i)),
        out_shape=jax.ShapeDtypeStruct((8, M), jnp.float32),
    )(z8, Wfd, bfd.reshape(1, M))
    return out[:1]


def _gat(x, src, dst, N, W, a_s, a_d, b):
    h = x @ W
    alpha_src = (h * a_s).sum(-1)
    alpha_dst = (h * a_d).sum(-1)
    alpha = jnp.take(alpha_src, src) + jnp.take(alpha_dst, dst)
    alpha = jax.nn.leaky_relu(alpha, 0.2)
    ex = jnp.exp(alpha)
    denom = jax.ops.segment_sum(ex, dst, num_segments=N)
    num = jax.ops.segment_sum(
        jnp.take(h, src, axis=0) * ex[:, None], dst, num_segments=N
    )
    return num / denom[:, None] + b


def kernel(x, edge_index, batch,
           W1e, as1e, ad1e, b1e, W2e, as2e, ad2e, b2e, W3e, as3e, ad3e, b3e,
           Wfe, bfe, Wfd, bfd,
           W1d, as1d, ad1d, b1d, W2d, as2d, ad2d, b2d, W3d, as3d, ad3d, b3d):
    N = x.shape[0]
    loops = jnp.arange(N, dtype=edge_index.dtype)
    src = jnp.concatenate([edge_index[0], loops])
    dst = jnp.concatenate([edge_index[1], loops])

    h = jax.nn.relu(_gat(x, src, dst, N, W1e, as1e, ad1e, b1e))
    h = jax.nn.relu(_gat(h, src, dst, N, W2e, as2e, ad2e, b2e))
    h = jax.nn.relu(_gat(h, src, dst, N, W3e, as3e, ad3e, b3e))
    g = jnp.max(h, axis=0, keepdims=True)
    z = g @ Wfe + bfe

    d = _fcd_matvec(z, Wfd, bfd)
    d = d.reshape(-1, 64)
    d = jax.nn.relu(d)
    d = jax.nn.relu(_gat(d, src, dst, N, W1d, as1d, ad1d, b1d))
    d = jax.nn.relu(_gat(d, src, dst, N, W2d, as2d, ad2d, b2d))
    d = jax.nn.sigmoid(_gat(d, src, dst, N, W3d, as3d, ad3d, b3d))
    return d


# speedup vs baseline: 24.4281x; 15.8119x over previous
import functools

import jax
import jax.numpy as jnp
from jax import lax
from jax.experimental import pallas as pl
from jax.experimental.pallas import tpu as pltpu
from jax.experimental.pallas import tpu_sc as plsc

_N = 10000
_NP = 10240
_E = 330000
_EW = 10368
_EP = _EW * 32
_NCH = _EW // 128


def _sweep_body(F, h_hbm, asrc_hbm, adst_hbm, src_hbm, dst_hbm, out_hbm,
                as_v, ad_v, src_v, dst_v, ex_v, rows_v, outb_v, zbuf_v,
                acc_sh, sem):
    FP = F + 16
    c = lax.axis_index("c")
    s = lax.axis_index("s")
    wid = c * 16 + s

    pltpu.sync_copy(asrc_hbm, as_v)
    pltpu.sync_copy(adst_hbm, ad_v)

    zero16 = jnp.zeros((16,), jnp.float32)
    for r in range(32):
        for k in range(FP // 16):
            zbuf_v[r, pl.ds(k * 16, 16)] = zero16
    rows_per_sub = _NP // 16

    def zeroacc(t, carry):
        pltpu.sync_copy(zbuf_v, acc_sh.at[pl.ds(s * rows_per_sub + t * 32, 32)])
        return carry

    lax.fori_loop(0, rows_per_sub // 32, zeroacc, 0)
    plsc.subcore_barrier()

    base = wid * _EW

    def chunk(j, carry):
        off = base + j * 128
        pltpu.sync_copy(src_hbm.at[pl.ds(off, 128)], src_v)
        pltpu.sync_copy(dst_hbm.at[pl.ds(off, 128)], dst_v)
        g = pltpu.async_copy(h_hbm.at[src_v], rows_v, sem)
        for i in range(8):
            s16 = src_v[pl.ds(i * 16, 16)]
            d16 = dst_v[pl.ds(i * 16, 16)]
            a = plsc.load_gather(as_v, [s16]) + plsc.load_gather(ad_v, [d16])
            a = jnp.maximum(a, 0.2 * a)
            ex_v[pl.ds(i * 16, 16)] = jnp.exp(a)
        g.wait()

        lane = lax.iota(jnp.int32, 16)

        def scale16(i, inner):
            ex16 = ex_v[pl.ds(i * 16, 16)]
            for l in range(16):
                e = i * 16 + l
                exs = ex16[l]
                for k in range(F // 16):
                    outb_v[e, pl.ds(k * 16, 16)] = (
                        rows_v[e, pl.ds(k * 16, 16)] * exs)
                outb_v[e, pl.ds(F, 16)] = jnp.where(lane == 0, exs, 0.0)
            return inner

        lax.fori_loop(0, 8, scale16, 0)
        pltpu.sync_copy(outb_v, acc_sh.at[dst_v], add=True)
        return carry

    lax.fori_loop(0, _NCH, chunk, 0)
    plsc.subcore_barrier()

    pltpu.sync_copy(acc_sh.at[pl.ds(s * rows_per_sub, rows_per_sub)],
                    out_hbm.at[c, pl.ds(s * rows_per_sub, rows_per_sub)])


@functools.lru_cache(maxsize=None)
def _make_sweep(F):
    FP = F + 16
    mesh = plsc.VectorSubcoreMesh(core_axis_name="c", subcore_axis_name="s")
    return pl.kernel(
        functools.partial(_sweep_body, F),
        out_type=jax.ShapeDtypeStruct((2, _NP, FP), jnp.float32),
        mesh=mesh,
        scratch_types=[
            pltpu.VMEM((_NP,), jnp.float32),
            pltpu.VMEM((_NP,), jnp.float32),
            pltpu.VMEM((128,), jnp.int32),
            pltpu.VMEM((128,), jnp.int32),
            pltpu.VMEM((128,), jnp.float32),
            pltpu.VMEM((128, F), jnp.float32),
            pltpu.VMEM((128, FP), jnp.float32),
            pltpu.VMEM((32, FP), jnp.float32),
            pltpu.VMEM_SHARED((_NP, FP), jnp.float32),
            pltpu.SemaphoreType.DMA,
        ],
        name=f"gat_sweep_f{F}",
        compiler_params=pltpu.CompilerParams(
            needs_layout_passes=False, use_tc_tiling_on_sc=False),
    )


def _gat_sc(x, srcp, dstp, W, a_s, a_d, b):
    F = W.shape[1]
    h = x @ W
    asrc = jnp.pad((h * a_s).sum(-1), (0, _NP - _N))
    adst = jnp.pad((h * a_d).sum(-1), (0, _NP - _N))
    CF = min(F, 64)
    parts, den = [], None
    for f0 in range(0, F, CF):
        acc = _make_sweep(CF)(h[:, f0:f0 + CF], asrc, adst, srcp, dstp)
        accs = acc[0, :_N] + acc[1, :_N]
        parts.append(accs[:, :CF])
        if den is None:
            den = accs[:, CF:CF + 1]
    num = jnp.concatenate(parts, axis=1) if len(parts) > 1 else parts[0]
    return num / den + b


def _fcd_body(z_ref, w_ref, b_ref, o_ref):
    o_ref[...] = (
        jnp.dot(z_ref[...], w_ref[...], preferred_element_type=jnp.float32)
        + b_ref[...]
    )


def _fcd_matvec(z, Wfd, bfd):
    K, M = Wfd.shape
    BC = 12800
    z8 = jnp.concatenate([z, jnp.zeros((7, K), z.dtype)], axis=0)
    out = pl.pallas_call(
        _fcd_body,
        grid=(M // BC,),
        in_specs=[
            pl.BlockSpec((8, K), lambda i: (0, 0)),
            pl.BlockSpec((K, BC), lambda i: (0, i)),
            pl.BlockSpec((1, BC), lambda i: (0, i)),
        ],
        out_specs=pl.BlockSpec((8, BC), lambda i: (0, i)),
        out_shape=jax.ShapeDtypeStruct((8, M), jnp.float32),
    )(z8, Wfd, bfd.reshape(1, M))
    return out[:1]


def kernel(x, edge_index, batch,
           W1e, as1e, ad1e, b1e, W2e, as2e, ad2e, b2e, W3e, as3e, ad3e, b3e,
           Wfe, bfe, Wfd, bfd,
           W1d, as1d, ad1d, b1d, W2d, as2d, ad2d, b2d, W3d, as3d, ad3d, b3d):
    N = x.shape[0]
    loops = jnp.arange(N, dtype=edge_index.dtype)
    src = jnp.concatenate([edge_index[0], loops])
    dst = jnp.concatenate([edge_index[1], loops])
    srcp = jnp.pad(src, (0, _EP - _E))
    dstp = jnp.pad(dst, (0, _EP - _E), constant_values=_N)

    h = jax.nn.relu(_gat_sc(x, srcp, dstp, W1e, as1e, ad1e, b1e))
    h = jax.nn.relu(_gat_sc(h, srcp, dstp, W2e, as2e, ad2e, b2e))
    h = jax.nn.relu(_gat_sc(h, srcp, dstp, W3e, as3e, ad3e, b3e))
    g = jnp.max(h, axis=0, keepdims=True)
    z = g @ Wfe + bfe

    d = _fcd_matvec(z, Wfd, bfd)
    d = d.reshape(-1, 64)
    d = jax.nn.relu(d)
    d = jax.nn.relu(_gat_sc(d, srcp, dstp, W1d, as1d, ad1d, b1d))
    d = jax.nn.relu(_gat_sc(d, srcp, dstp, W2d, as2d, ad2d, b2d))
    d = jax.nn.sigmoid(_gat_sc(d, srcp, dstp, W3d, as3d, ad3d, b3d))
    return d


# speedup vs baseline: 37.4364x; 1.5325x over previous
import functools

import jax
import jax.numpy as jnp
from jax import lax
from jax.experimental import pallas as pl
from jax.experimental.pallas import tpu as pltpu
from jax.experimental.pallas import tpu_sc as plsc

_N = 10000
_NP = 10016
_E = 330000
_EW = 10496
_EP = _EW * 32
_NCH = _EW // 128


def _sweep_body(F, h_hbm, asrc_hbm, adst_hbm, src_hbm, dst_hbm, out_hbm,
                as_v, ad_v, src_v, dst_v, ex0_v, ex1_v, rows0_v, rows1_v,
                outb0_v, outb1_v, zbuf_v, acc_sh, g0, g1, s0, s1):
    FP = F + 16
    c = lax.axis_index("c")
    s = lax.axis_index("s")
    wid = c * 16 + s

    pltpu.sync_copy(asrc_hbm, as_v)
    pltpu.sync_copy(adst_hbm, ad_v)
    pltpu.sync_copy(src_hbm.at[pl.ds(wid * _NCH, _NCH)], src_v)
    pltpu.sync_copy(dst_hbm.at[pl.ds(wid * _NCH, _NCH)], dst_v)

    zero16 = jnp.zeros((16,), jnp.float32)
    for r in range(32):
        for k in range(FP // 16):
            zbuf_v[r, pl.ds(k * 16, 16)] = zero16
    rows_per_sub = _NP // 16

    def zeroacc(t, carry):
        pltpu.sync_copy(zbuf_v, acc_sh.at[pl.ds(s * rows_per_sub + t * 32, 32)])
        return carry

    lax.fori_loop(0, rows_per_sub // 32, zeroacc, 0)
    rem = rows_per_sub % 32
    if rem:
        pltpu.sync_copy(
            zbuf_v.at[pl.ds(0, rem)],
            acc_sh.at[pl.ds(s * rows_per_sub + rows_per_sub - rem, rem)])
    plsc.subcore_barrier()

    lane = lax.iota(jnp.int32, 16)

    def compute_ex(j, ex_v):
        for i in range(8):
            s16 = src_v[j, pl.ds(i * 16, 16)]
            d16 = dst_v[j, pl.ds(i * 16, 16)]
            a = plsc.load_gather(as_v, [s16]) + plsc.load_gather(ad_v, [d16])
            a = jnp.maximum(a, 0.2 * a)
            ex_v[pl.ds(i * 16, 16)] = jnp.exp(a)

    def scale(rows_v, ex_v, outb_v):
        def scale16(i, inner):
            ex16 = ex_v[pl.ds(i * 16, 16)]
            for l in range(16):
                e = i * 16 + l
                exs = ex16[l]
                for k in range(F // 16):
                    outb_v[e, pl.ds(k * 16, 16)] = (
                        rows_v[e, pl.ds(k * 16, 16)] * exs)
                outb_v[e, pl.ds(F, 16)] = jnp.where(lane == 0, exs, 0.0)
            return inner

        lax.fori_loop(0, 8, scale16, 0)

    def issue_gather(j, rows_v, sem):
        return pltpu.async_copy(h_hbm.at[src_v.at[j]], rows_v, sem)

    def drain_gather(rows_v, sem):
        pltpu.make_async_copy(h_hbm.at[src_v.at[0]], rows_v, sem).wait()

    def issue_scatter(outb_v, j, sem):
        pltpu.async_copy(outb_v, acc_sh.at[dst_v.at[j]], sem, add=True)

    def drain_scatter(outb_v, sem):
        pltpu.make_async_copy(outb_v, acc_sh.at[dst_v.at[0]], sem).wait()

    def do_pair(jj, first):
        j0 = jj * 2
        j1 = j0 + 1
        compute_ex(j0, ex0_v)
        gd1 = issue_gather(j1, rows1_v, g1)
        drain_gather(rows0_v, g0)
        if not first:
            drain_scatter(outb0_v, s0)
        scale(rows0_v, ex0_v, outb0_v)
        issue_scatter(outb0_v, j0, s0)
        compute_ex(j1, ex1_v)
        if first:
            issue_gather(j0 + 2, rows0_v, g0)
        else:
            @pl.when(jj < _NCH // 2 - 1)
            def _():
                issue_gather(j0 + 2, rows0_v, g0)
        gd1.wait()
        if not first:
            drain_scatter(outb1_v, s1)
        scale(rows1_v, ex1_v, outb1_v)
        issue_scatter(outb1_v, j1, s1)

    issue_gather(0, rows0_v, g0)
    do_pair(0, True)

    def pair_body(jj, carry):
        do_pair(jj, False)
        return carry

    lax.fori_loop(1, _NCH // 2, pair_body, 0)
    drain_scatter(outb0_v, s0)
    drain_scatter(outb1_v, s1)
    plsc.subcore_barrier()

    pltpu.sync_copy(acc_sh.at[pl.ds(s * rows_per_sub, rows_per_sub)],
                    out_hbm.at[c, pl.ds(s * rows_per_sub, rows_per_sub)])


@functools.lru_cache(maxsize=None)
def _make_sweep(F):
    FP = F + 16
    mesh = plsc.VectorSubcoreMesh(core_axis_name="c", subcore_axis_name="s")
    return pl.kernel(
        functools.partial(_sweep_body, F),
        out_type=jax.ShapeDtypeStruct((2, _NP, FP), jnp.float32),
        mesh=mesh,
        scratch_types=[
            pltpu.VMEM((_NP,), jnp.float32),
            pltpu.VMEM((_NP,), jnp.float32),
            pltpu.VMEM((_NCH, 128), jnp.int32),
            pltpu.VMEM((_NCH, 128), jnp.int32),
            pltpu.VMEM((128,), jnp.float32),
            pltpu.VMEM((128,), jnp.float32),
            pltpu.VMEM((128, F), jnp.float32),
            pltpu.VMEM((128, F), jnp.float32),
            pltpu.VMEM((128, FP), jnp.float32),
            pltpu.VMEM((128, FP), jnp.float32),
            pltpu.VMEM((32, FP), jnp.float32),
            pltpu.VMEM_SHARED((_NP, FP), jnp.float32),
            pltpu.SemaphoreType.DMA,
            pltpu.SemaphoreType.DMA,
            pltpu.SemaphoreType.DMA,
            pltpu.SemaphoreType.DMA,
        ],
        name=f"gat_sweep_f{F}",
        compiler_params=pltpu.CompilerParams(
            needs_layout_passes=False, use_tc_tiling_on_sc=False),
    )


def _gat_sc(x, srcp, dstp, W, a_s, a_d, b):
    F = W.shape[1]
    h = x @ W
    asrc = jnp.pad((h * a_s).sum(-1), (0, _NP - _N))
    adst = jnp.pad((h * a_d).sum(-1), (0, _NP - _N))
    CF = min(F, 64)
    parts, den = [], None
    for f0 in range(0, F, CF):
        acc = _make_sweep(CF)(h[:, f0:f0 + CF], asrc, adst, srcp, dstp)
        accs = acc[0, :_N] + acc[1, :_N]
        parts.append(accs[:, :CF])
        if den is None:
            den = accs[:, CF:CF + 1]
    num = jnp.concatenate(parts, axis=1) if len(parts) > 1 else parts[0]
    return num / den + b


def _fcd_body(z_ref, w_ref, b_ref, o_ref):
    o_ref[...] = (
        jnp.dot(z_ref[...], w_ref[...], preferred_element_type=jnp.float32)
        + b_ref[...]
    )


def _fcd_matvec(z, Wfd, bfd):
    K, M = Wfd.shape
    BC = 12800
    z8 = jnp.concatenate([z, jnp.zeros((7, K), z.dtype)], axis=0)
    out = pl.pallas_call(
        _fcd_body,
        grid=(M // BC,),
        in_specs=[
            pl.BlockSpec((8, K), lambda i: (0, 0)),
            pl.BlockSpec((K, BC), lambda i: (0, i)),
            pl.BlockSpec((1, BC), lambda i: (0, i)),
        ],
        out_specs=pl.BlockSpec((8, BC), lambda i: (0, i)),
        out_shape=jax.ShapeDtypeStruct((8, M), jnp.float32),
    )(z8, Wfd, bfd.reshape(1, M))
    return out[:1]


def kernel(x, edge_index, batch,
           W1e, as1e, ad1e, b1e, W2e, as2e, ad2e, b2e, W3e, as3e, ad3e, b3e,
           Wfe, bfe, Wfd, bfd,
           W1d, as1d, ad1d, b1d, W2d, as2d, ad2d, b2d, W3d, as3d, ad3d, b3d):
    N = x.shape[0]
    loops = jnp.arange(N, dtype=edge_index.dtype)
    src = jnp.concatenate([edge_index[0], loops])
    dst = jnp.concatenate([edge_index[1], loops])
    srcp = jnp.pad(src, (0, _EP - _E)).reshape(_EP // 128, 128)
    dstp = jnp.pad(dst, (0, _EP - _E), constant_values=_N).reshape(
        _EP // 128, 128)

    h = jax.nn.relu(_gat_sc(x, srcp, dstp, W1e, as1e, ad1e, b1e))
    h = jax.nn.relu(_gat_sc(h, srcp, dstp, W2e, as2e, ad2e, b2e))
    h = jax.nn.relu(_gat_sc(h, srcp, dstp, W3e, as3e, ad3e, b3e))
    g = jnp.max(h, axis=0, keepdims=True)
    z = g @ Wfe + bfe

    d = _fcd_matvec(z, Wfd, bfd)
    d = d.reshape(-1, 64)
    d = jax.nn.relu(d)
    d = jax.nn.relu(_gat_sc(d, srcp, dstp, W1d, as1d, ad1d, b1d))
    d = jax.nn.relu(_gat_sc(d, srcp, dstp, W2d, as2d, ad2d, b2d))
    d = jax.nn.sigmoid(_gat_sc(d, srcp, dstp, W3d, as3d, ad3d, b3d))
    return d
